# Initial kernel scaffold; baseline (speedup 1.0000x reference)
import jax
import jax.numpy as jnp
from jax.experimental import pallas as pl


def kernel(x, edge_index, batch, device, temp, thr_param):
    raise NotImplementedError("write your pallas kernel here")


# speedup vs baseline: 3.7166x; 3.7166x over previous
import functools

import jax
import jax.numpy as jnp
from jax import lax
from jax.experimental import pallas as pl
from jax.experimental.pallas import tpu as pltpu
from jax.experimental.pallas import tpu_sc as plsc

N_NODES = 10000
N_EDGES = 320000
D_FEAT = 128

NC = 2
NS = 16
NW = NC * NS
E_PER_W = N_EDGES // NW
C = 80
NCHUNK = E_PER_W // C
L = 16


def _affinity_body(x_hbm, row_hbm, col_hbm, temp_hbm,
                   aff_hbm, part_hbm,
                   idx_r, idx_c, rows_v, cols_v, aff_buf, temp_v, loss_v,
                   stage, sem):
    wid = lax.axis_index("s") * NC + lax.axis_index("c")
    base0 = wid * E_PER_W

    pltpu.sync_copy(row_hbm.at[wid], idx_r)
    pltpu.sync_copy(col_hbm.at[wid], idx_c)
    pltpu.sync_copy(temp_hbm, temp_v)

    zeros = jnp.zeros((L,), jnp.float32)
    loss_v[...] = zeros
    tv = temp_v[...]

    def chunk_body(ci, carry):
        cp1 = pltpu.async_copy(x_hbm.at[idx_r.at[ci]], rows_v, sem)
        cp2 = pltpu.async_copy(x_hbm.at[idx_c.at[ci]], cols_v, sem)
        cp1.wait()
        cp2.wait()

        lanes = lax.iota(jnp.int32, L)
        last_lane = lanes == (L - 1)

        def group_body(g, lacc):
            for j in range(L):
                e = g * L + j
                acc = rows_v[e, pl.ds(0, L)] * cols_v[e, pl.ds(0, L)]
                for k in range(1, D_FEAT // L):
                    acc = acc + (rows_v[e, pl.ds(k * L, L)]
                                 * cols_v[e, pl.ds(k * L, L)])
                cum = plsc.cumsum(acc)
                plsc.store_scatter(stage, [jnp.full((L,), j, jnp.int32)], cum,
                                   mask=last_lane)
            dots = stage[...]
            z = dots * tv
            a = 1.0 / (1.0 + jnp.exp(-z))
            aff_buf[pl.ds(g * L, L)] = a
            return lacc + jnp.minimum(a, 1.0 - a)

        lacc = lax.fori_loop(0, C // L, group_body, carry)
        pltpu.sync_copy(aff_buf, aff_hbm.at[pl.ds(base0 + ci * C, C)])
        return lacc

    loss = lax.fori_loop(0, NCHUNK, chunk_body, zeros)
    loss_v[...] = loss
    pltpu.sync_copy(loss_v, part_hbm.at[wid])


@jax.jit
def _affinity_sc(x, row3, col3, temp_v):
    mesh = plsc.VectorSubcoreMesh(core_axis_name="c", subcore_axis_name="s",
                                  num_cores=NC, num_subcores=NS)
    run = pl.kernel(
        _affinity_body,
        out_type=[
            jax.ShapeDtypeStruct((N_EDGES,), jnp.float32),
            jax.ShapeDtypeStruct((NW, L), jnp.float32),
        ],
        mesh=mesh,
        scratch_types=[
            pltpu.VMEM((NCHUNK, C), jnp.int32),
            pltpu.VMEM((NCHUNK, C), jnp.int32),
            pltpu.VMEM((C, D_FEAT), jnp.float32),
            pltpu.VMEM((C, D_FEAT), jnp.float32),
            pltpu.VMEM((C,), jnp.float32),
            pltpu.VMEM((L,), jnp.float32),
            pltpu.VMEM((L,), jnp.float32),
            pltpu.VMEM((L,), jnp.float32),
            pltpu.SemaphoreType.DMA,
        ],
        compiler_params=pltpu.CompilerParams(needs_layout_passes=False),
    )
    return run(x, row3, col3, temp_v)


def kernel(x, edge_index, batch, device, temp, thr_param):
    row3 = edge_index[0].reshape(NW, NCHUNK, C)
    col3 = edge_index[1].reshape(NW, NCHUNK, C)
    temp_v = jnp.broadcast_to(temp.astype(jnp.float32), (L,))
    affinities, partials = _affinity_sc(x, row3, col3, temp_v)
    threshold = jax.nn.sigmoid(thr_param)
    losses = jnp.sum(partials) / N_EDGES
    return (affinities, threshold, losses)


# speedup vs baseline: 6.3548x; 1.7099x over previous
import jax
import jax.numpy as jnp
from jax import lax
from jax.experimental import pallas as pl
from jax.experimental.pallas import tpu as pltpu
from jax.experimental.pallas import tpu_sc as plsc

N_NODES = 10000
N_EDGES = 320000
D_FEAT = 128

NC = 2
NS = 16
NW = NC * NS
E_PER_W = N_EDGES // NW
C = 80
NCHUNK = E_PER_W // C
L = 16


def _affinity_body(x_hbm, row_hbm, col_hbm, temp_hbm,
                   aff_hbm, part_hbm,
                   idx_r, idx_c, rows0, cols0, rows1, cols1,
                   aff0, aff1, temp_v, loss_v, stage,
                   gsem0, gsem1, osem0, osem1):
    wid = lax.axis_index("s") * NC + lax.axis_index("c")
    base0 = wid * E_PER_W

    pltpu.sync_copy(row_hbm.at[wid], idx_r)
    pltpu.sync_copy(col_hbm.at[wid], idx_c)
    pltpu.sync_copy(temp_hbm, temp_v)

    zeros = jnp.zeros((L,), jnp.float32)
    tv = temp_v[...]
    lanes = lax.iota(jnp.int32, L)
    last_lane = lanes == (L - 1)

    def fire(ci, rows, cols, sem):
        pltpu.async_copy(x_hbm.at[idx_r.at[ci]], rows, sem)
        pltpu.async_copy(x_hbm.at[idx_c.at[ci]], cols, sem)

    def wait_gather(ci, rows, cols, sem):
        pltpu.make_async_copy(x_hbm.at[idx_r.at[ci]], rows, sem).wait()
        pltpu.make_async_copy(x_hbm.at[idx_c.at[ci]], cols, sem).wait()

    def fire_store(ci, aff, sem):
        pltpu.async_copy(aff, aff_hbm.at[pl.ds(base0 + ci * C, C)], sem)

    def wait_store(ci, aff, sem):
        pltpu.make_async_copy(aff, aff_hbm.at[pl.ds(base0 + ci * C, C)],
                              sem).wait()

    def compute(rows, cols, aff, carry):
        def group_body(g, lacc):
            for j in range(L):
                e = g * L + j
                acc = rows[e, pl.ds(0, L)] * cols[e, pl.ds(0, L)]
                for k in range(1, D_FEAT // L):
                    acc = acc + (rows[e, pl.ds(k * L, L)]
                                 * cols[e, pl.ds(k * L, L)])
                cum = plsc.cumsum(acc)
                plsc.store_scatter(stage, [jnp.full((L,), j, jnp.int32)], cum,
                                   mask=last_lane)
            dots = stage[...]
            z = dots * tv
            a = 1.0 / (1.0 + jnp.exp(-z))
            aff[pl.ds(g * L, L)] = a
            return lacc + jnp.minimum(a, 1.0 - a)

        return lax.fori_loop(0, C // L, group_body, carry)

    fire(0, rows0, cols0, gsem0)

    def pair_body(i, carry):
        ci0 = 2 * i
        ci1 = 2 * i + 1
        fire(ci1, rows1, cols1, gsem1)
        wait_gather(ci0, rows0, cols0, gsem0)

        @pl.when(i > 0)
        def _():
            wait_store(ci0 - 2, aff0, osem0)

        carry = compute(rows0, cols0, aff0, carry)
        fire_store(ci0, aff0, osem0)
        fire(ci0 + 2, rows0, cols0, gsem0)

        wait_gather(ci1, rows1, cols1, gsem1)

        @pl.when(i > 0)
        def _():
            wait_store(ci1 - 2, aff1, osem1)

        carry = compute(rows1, cols1, aff1, carry)
        fire_store(ci1, aff1, osem1)
        return carry

    loss = lax.fori_loop(0, NCHUNK // 2, pair_body, zeros)

    last = NCHUNK - 1
    wait_gather(last, rows0, cols0, gsem0)
    wait_store(last - 2, aff0, osem0)
    loss = compute(rows0, cols0, aff0, loss)
    fire_store(last, aff0, osem0)

    wait_store(last - 1, aff1, osem1)
    wait_store(last, aff0, osem0)

    loss_v[...] = loss
    pltpu.sync_copy(loss_v, part_hbm.at[wid])


@jax.jit
def _affinity_sc(x, row3, col3, temp_v):
    mesh = plsc.VectorSubcoreMesh(core_axis_name="c", subcore_axis_name="s",
                                  num_cores=NC, num_subcores=NS)
    run = pl.kernel(
        _affinity_body,
        out_type=[
            jax.ShapeDtypeStruct((N_EDGES,), jnp.float32),
            jax.ShapeDtypeStruct((NW, L), jnp.float32),
        ],
        mesh=mesh,
        scratch_types=[
            pltpu.VMEM((NCHUNK, C), jnp.int32),
            pltpu.VMEM((NCHUNK, C), jnp.int32),
            pltpu.VMEM((C, D_FEAT), jnp.float32),
            pltpu.VMEM((C, D_FEAT), jnp.float32),
            pltpu.VMEM((C, D_FEAT), jnp.float32),
            pltpu.VMEM((C, D_FEAT), jnp.float32),
            pltpu.VMEM((C,), jnp.float32),
            pltpu.VMEM((C,), jnp.float32),
            pltpu.VMEM((L,), jnp.float32),
            pltpu.VMEM((L,), jnp.float32),
            pltpu.VMEM((L,), jnp.float32),
            pltpu.SemaphoreType.DMA,
            pltpu.SemaphoreType.DMA,
            pltpu.SemaphoreType.DMA,
            pltpu.SemaphoreType.DMA,
        ],
        compiler_params=pltpu.CompilerParams(needs_layout_passes=False),
    )
    return run(x, row3, col3, temp_v)


def kernel(x, edge_index, batch, device, temp, thr_param):
    row3 = edge_index[0].reshape(NW, NCHUNK, C)
    col3 = edge_index[1].reshape(NW, NCHUNK, C)
    temp_v = jnp.broadcast_to(temp.astype(jnp.float32), (L,))
    affinities, partials = _affinity_sc(x, row3, col3, temp_v)
    threshold = jax.nn.sigmoid(thr_param)
    losses = jnp.sum(partials) / N_EDGES
    return (affinities, threshold, losses)
